# baseline (device time: 33509 ns/iter reference)
import jax
import jax.numpy as jnp
from jax import lax
from jax.experimental import pallas as pl
from jax.experimental.pallas import tpu as pltpu

B, H, D, BS = 8, 8, 128, 16
NPAGES_LOCAL = 512
NPB = 64
NBLK = NPAGES_LOCAL // NPB // 2
T = NPB * BS
NEG = -1e30
SCALE = D ** -0.5


def _merge_exchange(stage, nbr, acc_ref, m_ref, l_ref,
                    acc_comm, ml_send, ml_comm, send_sems, recv_sems):
    s_a, s_m = 2 * stage, 2 * stage + 1
    ml_send[0, :, :] = m_ref[...]
    ml_send[1, :, :] = l_ref[...]
    rdma_acc = pltpu.make_async_remote_copy(
        src_ref=acc_ref, dst_ref=acc_comm,
        send_sem=send_sems.at[s_a], recv_sem=recv_sems.at[s_a],
        device_id=nbr, device_id_type=pl.DeviceIdType.MESH)
    rdma_ml = pltpu.make_async_remote_copy(
        src_ref=ml_send, dst_ref=ml_comm,
        send_sem=send_sems.at[s_m], recv_sem=recv_sems.at[s_m],
        device_id=nbr, device_id_type=pl.DeviceIdType.MESH)
    rdma_acc.start()
    rdma_ml.start()
    rdma_acc.wait()
    rdma_ml.wait()

    m_loc = m_ref[...]
    m_rem = ml_comm[0, :, :]
    l_rem = ml_comm[1, :, :]
    m_f = jnp.maximum(m_loc, m_rem)
    a_loc = jnp.exp(m_loc - m_f)
    a_rem = jnp.exp(m_rem - m_f)
    m_ref[...] = m_f
    l_ref[...] = a_loc * l_ref[...] + a_rem * l_rem
    acc_ref[...] = (a_loc[:, :, None] * acc_ref[...]
                    + a_rem[:, :, None] * acc_comm[...])


def _body(x_off_ref, q_ref, k_ref, v_ref, btv_ref, out_ref,
          acc_ref, m_ref, l_ref,
          acc_comm, ml_send, ml_comm, send_sems, recv_sems):
    i = pl.program_id(0)
    my_x = lax.axis_index("x")
    my_y = lax.axis_index("y")
    nbr_y = (my_x, 1 - my_y)
    nbr_x = (1 - my_x, my_y)

    @pl.when(i == 0)
    def _init():
        barrier = pltpu.get_barrier_semaphore()
        for nbr in (nbr_y, nbr_x):
            pl.semaphore_signal(barrier, inc=1, device_id=nbr,
                                device_id_type=pl.DeviceIdType.MESH)
        pl.semaphore_wait(barrier, 2)
        m_ref[...] = jnp.full((H, B), NEG, jnp.float32)
        l_ref[...] = jnp.zeros((H, B), jnp.float32)
        acc_ref[...] = jnp.zeros((H, B, D), jnp.float32)

    base = my_y * NPAGES_LOCAL + (my_x * NBLK + i) * NPB
    pidc = base + lax.broadcasted_iota(jnp.int32, (1, NPB, 1), 1)
    btv = btv_ref[...]
    counts = jnp.sum((btv[:, None, :] == pidc).astype(jnp.float32),
                     axis=2)
    w = jnp.broadcast_to(counts[:, :, None], (B, NPB, BS)).reshape(B, T)

    q = q_ref[...].reshape(B, H, D).astype(jnp.bfloat16)
    kr = k_ref[...].reshape(T, H, D).astype(jnp.bfloat16)
    vr = v_ref[...].reshape(T, H, D).astype(jnp.bfloat16)

    s_list = []
    for h in range(H):
        s_list.append(lax.dot_general(
            q[:, h, :], kr[:, h, :],
            (((1,), (1,)), ((), ())),
            preferred_element_type=jnp.float32))
    s = jnp.stack(s_list, axis=0) * SCALE
    s = jnp.where((w > 0.0)[None, :, :], s, NEG)

    m_prev = m_ref[...]
    m_new = jnp.maximum(m_prev, jnp.max(s, axis=2))
    alpha = jnp.exp(m_prev - m_new)
    p = jnp.exp(s - m_new[:, :, None]) * w[None, :, :]
    m_ref[...] = m_new
    l_ref[...] = alpha * l_ref[...] + jnp.sum(p, axis=2)

    pb = p.astype(jnp.bfloat16)
    o_list = []
    for h in range(H):
        o_list.append(lax.dot_general(
            pb[h], vr[:, h, :],
            (((1,), (0,)), ((), ())),
            preferred_element_type=jnp.float32))
    o = jnp.stack(o_list, axis=0)
    acc_ref[...] = alpha[:, :, None] * acc_ref[...] + o

    @pl.when(i == NBLK - 1)
    def _finish():
        _merge_exchange(0, nbr_y, acc_ref, m_ref, l_ref,
                        acc_comm, ml_send, ml_comm, send_sems, recv_sems)
        _merge_exchange(1, nbr_x, acc_ref, m_ref, l_ref,
                        acc_comm, ml_send, ml_comm, send_sems, recv_sems)

        out = acc_ref[...] / l_ref[...][:, :, None]
        out_ref[...] = jnp.transpose(out, (1, 0, 2)).reshape(B, 1, H, D)


def kernel(Q, K, V, bt, lens):
    jidx = lax.broadcasted_iota(jnp.int32, (B, NPAGES_LOCAL), 1)
    btv = jnp.where(jidx < lens[:, None], bt, -1)

    x_off = (lax.axis_index("x") * NBLK).astype(jnp.int32).reshape(1)

    grid_spec = pltpu.PrefetchScalarGridSpec(
        num_scalar_prefetch=1,
        grid=(NBLK,),
        in_specs=[
            pl.BlockSpec((B, 1, H, D), lambda i, xo: (0, 0, 0, 0)),
            pl.BlockSpec((NPB, BS, H, D), lambda i, xo: (xo[0] + i, 0, 0, 0)),
            pl.BlockSpec((NPB, BS, H, D), lambda i, xo: (xo[0] + i, 0, 0, 0)),
            pl.BlockSpec((B, NPAGES_LOCAL), lambda i, xo: (0, 0)),
        ],
        out_specs=pl.BlockSpec((B, 1, H, D), lambda i, xo: (0, 0, 0, 0)),
        scratch_shapes=[
            pltpu.VMEM((H, B, D), jnp.float32),
            pltpu.VMEM((H, B), jnp.float32),
            pltpu.VMEM((H, B), jnp.float32),
            pltpu.VMEM((H, B, D), jnp.float32),
            pltpu.VMEM((2, H, B), jnp.float32),
            pltpu.VMEM((2, H, B), jnp.float32),
            pltpu.SemaphoreType.DMA((4,)),
            pltpu.SemaphoreType.DMA((4,)),
        ],
    )

    return pl.pallas_call(
        _body,
        grid_spec=grid_spec,
        out_shape=jax.ShapeDtypeStruct((B, 1, H, D), jnp.float32),
        compiler_params=pltpu.CompilerParams(
            collective_id=0,
            dimension_semantics=("arbitrary",),
        ),
    )(x_off, Q, K, V, btv)


# device time: 31082 ns/iter; 1.0781x vs baseline; 1.0781x over previous
import jax
import jax.numpy as jnp
from jax import lax
from jax.experimental import pallas as pl
from jax.experimental.pallas import tpu as pltpu

B, H, D, BS = 8, 8, 128, 16
HB = H * B
NPAGES_LOCAL = 512
NPB = 32
NBLK = NPAGES_LOCAL // NPB // 2
T = NPB * BS
NEG = -1e30
SCALE = D ** -0.5


def _merge_exchange(stage, nbr, acc_ref, m_ref, l_ref,
                    acc_comm, ml_send, ml_comm, send_sems, recv_sems):
    s_a, s_m = 2 * stage, 2 * stage + 1
    ml_send[0, :, :] = m_ref[...]
    ml_send[1, :, :] = l_ref[...]
    rdma_acc = pltpu.make_async_remote_copy(
        src_ref=acc_ref, dst_ref=acc_comm,
        send_sem=send_sems.at[s_a], recv_sem=recv_sems.at[s_a],
        device_id=nbr, device_id_type=pl.DeviceIdType.MESH)
    rdma_ml = pltpu.make_async_remote_copy(
        src_ref=ml_send, dst_ref=ml_comm,
        send_sem=send_sems.at[s_m], recv_sem=recv_sems.at[s_m],
        device_id=nbr, device_id_type=pl.DeviceIdType.MESH)
    rdma_acc.start()
    rdma_ml.start()
    rdma_acc.wait()
    rdma_ml.wait()

    m_loc = m_ref[...]
    m_rem = ml_comm[0, :, :]
    l_rem = ml_comm[1, :, :]
    m_f = jnp.maximum(m_loc, m_rem)
    a_loc = jnp.exp(m_loc - m_f)
    a_rem = jnp.exp(m_rem - m_f)
    m_ref[...] = m_f
    l_ref[...] = a_loc * l_ref[...] + a_rem * l_rem
    acc_ref[...] = a_loc * acc_ref[...] + a_rem * acc_comm[...]


def _body(x_off_ref, q_ref, k_ref, v_ref, btv_ref, out_ref,
          acc_ref, m_ref, l_ref,
          acc_comm, ml_send, ml_comm, send_sems, recv_sems):
    i = pl.program_id(0)
    my_x = lax.axis_index("x")
    my_y = lax.axis_index("y")
    nbr_y = (my_x, 1 - my_y)
    nbr_x = (1 - my_x, my_y)

    @pl.when(i == 0)
    def _init():
        barrier = pltpu.get_barrier_semaphore()
        for nbr in (nbr_y, nbr_x):
            pl.semaphore_signal(barrier, inc=1, device_id=nbr,
                                device_id_type=pl.DeviceIdType.MESH)
        pl.semaphore_wait(barrier, 2)
        m_ref[...] = jnp.full((HB, 1), NEG, jnp.float32)
        l_ref[...] = jnp.zeros((HB, 1), jnp.float32)
        acc_ref[...] = jnp.zeros((HB, D), jnp.float32)

    base = my_y * NPAGES_LOCAL + (my_x * NBLK + i) * NPB
    pidc = base + lax.broadcasted_iota(jnp.int32, (1, NPB, 1), 1)
    btv = btv_ref[...]
    counts = jnp.sum((btv[:, None, :] == pidc).astype(jnp.float32),
                     axis=2)
    expand = (lax.broadcasted_iota(jnp.int32, (NPB, T), 1) // BS
              == lax.broadcasted_iota(jnp.int32, (NPB, T), 0)
              ).astype(jnp.float32)
    w = lax.dot_general(counts, expand, (((1,), (0,)), ((), ())),
                        preferred_element_type=jnp.float32)
    w_log = jnp.where(w > 0.5, jnp.log(w), NEG)

    q2 = q_ref[...]
    kr = k_ref[...].reshape(T, H, D)
    vr = v_ref[...].reshape(T, H, D)

    s_list = []
    for h in range(H):
        s_h = lax.dot_general(
            q2[h * B:(h + 1) * B, :], kr[:, h, :],
            (((1,), (1,)), ((), ())),
            preferred_element_type=jnp.float32)
        s_list.append(s_h * SCALE + w_log)
    s2 = jnp.concatenate(s_list, axis=0)

    m_prev = m_ref[...]
    m_new = jnp.maximum(m_prev, jnp.max(s2, axis=1, keepdims=True))
    alpha = jnp.exp(m_prev - m_new)
    p2 = jnp.exp(s2 - m_new)
    m_ref[...] = m_new
    l_ref[...] = alpha * l_ref[...] + jnp.sum(p2, axis=1, keepdims=True)

    o_list = []
    for h in range(H):
        o_list.append(lax.dot_general(
            p2[h * B:(h + 1) * B, :], vr[:, h, :],
            (((1,), (0,)), ((), ())),
            preferred_element_type=jnp.float32))
    o2 = jnp.concatenate(o_list, axis=0)
    acc_ref[...] = alpha * acc_ref[...] + o2

    @pl.when(i == NBLK - 1)
    def _finish():
        _merge_exchange(0, nbr_y, acc_ref, m_ref, l_ref,
                        acc_comm, ml_send, ml_comm, send_sems, recv_sems)
        _merge_exchange(1, nbr_x, acc_ref, m_ref, l_ref,
                        acc_comm, ml_send, ml_comm, send_sems, recv_sems)

        out = (acc_ref[...] / l_ref[...]).reshape(H, B, D)
        out_ref[...] = jnp.transpose(out, (1, 0, 2)).reshape(B, 1, H, D)


def kernel(Q, K, V, bt, lens):
    jidx = lax.broadcasted_iota(jnp.int32, (B, NPAGES_LOCAL), 1)
    btv = jnp.where(jidx < lens[:, None], bt, -1)

    q2 = jnp.transpose(Q.reshape(B, H, D), (1, 0, 2)).reshape(HB, D)

    x_off = (lax.axis_index("x") * NBLK).astype(jnp.int32).reshape(1)

    grid_spec = pltpu.PrefetchScalarGridSpec(
        num_scalar_prefetch=1,
        grid=(NBLK,),
        in_specs=[
            pl.BlockSpec((HB, D), lambda i, xo: (0, 0)),
            pl.BlockSpec((NPB, BS, H, D), lambda i, xo: (xo[0] + i, 0, 0, 0)),
            pl.BlockSpec((NPB, BS, H, D), lambda i, xo: (xo[0] + i, 0, 0, 0)),
            pl.BlockSpec((B, NPAGES_LOCAL), lambda i, xo: (0, 0)),
        ],
        out_specs=pl.BlockSpec((B, 1, H, D), lambda i, xo: (0, 0, 0, 0)),
        scratch_shapes=[
            pltpu.VMEM((HB, D), jnp.float32),
            pltpu.VMEM((HB, 1), jnp.float32),
            pltpu.VMEM((HB, 1), jnp.float32),
            pltpu.VMEM((HB, D), jnp.float32),
            pltpu.VMEM((2, HB, 1), jnp.float32),
            pltpu.VMEM((2, HB, 1), jnp.float32),
            pltpu.SemaphoreType.DMA((4,)),
            pltpu.SemaphoreType.DMA((4,)),
        ],
    )

    return pl.pallas_call(
        _body,
        grid_spec=grid_spec,
        out_shape=jax.ShapeDtypeStruct((B, 1, H, D), jnp.float32),
        compiler_params=pltpu.CompilerParams(
            collective_id=0,
            dimension_semantics=("arbitrary",),
        ),
    )(x_off, q2, K, V, btv)


# device time: 30299 ns/iter; 1.1059x vs baseline; 1.0258x over previous
import jax
import jax.numpy as jnp
from jax import lax
from jax.experimental import pallas as pl
from jax.experimental.pallas import tpu as pltpu

B, H, D, BS = 8, 8, 128, 16
HB = H * B
NPAGES_LOCAL = 512
NPB = 64
NBLK = NPAGES_LOCAL // NPB // 2
T = NPB * BS
NEG = -1e30
SCALE = D ** -0.5


def _body(x_off_ref, q_ref, k_ref, v_ref, btv_ref, out_ref,
          acc_ref, m_ref, l_ref,
          acc_comm, ml_send, ml_comm, send_sems, recv_sems):
    i = pl.program_id(0)
    my_x = lax.axis_index("x")
    my_y = lax.axis_index("y")
    peers = ((my_x, 1 - my_y), (1 - my_x, my_y), (1 - my_x, 1 - my_y))

    @pl.when(i == 0)
    def _init():
        barrier = pltpu.get_barrier_semaphore()
        for nbr in peers:
            pl.semaphore_signal(barrier, inc=1, device_id=nbr,
                                device_id_type=pl.DeviceIdType.MESH)
        pl.semaphore_wait(barrier, 3)
        m_ref[...] = jnp.full((HB, 1), NEG, jnp.float32)
        l_ref[...] = jnp.zeros((HB, 1), jnp.float32)
        acc_ref[...] = jnp.zeros((HB, D), jnp.float32)

    base = my_y * NPAGES_LOCAL + (my_x * NBLK + i) * NPB
    pidc = base + lax.broadcasted_iota(jnp.int32, (1, NPB, 1), 1)
    btv = btv_ref[...]
    counts = jnp.sum((btv[:, None, :] == pidc).astype(jnp.float32),
                     axis=2)
    expand = (lax.broadcasted_iota(jnp.int32, (NPB, T), 1) // BS
              == lax.broadcasted_iota(jnp.int32, (NPB, T), 0)
              ).astype(jnp.float32)
    w = lax.dot_general(counts, expand, (((1,), (0,)), ((), ())),
                        preferred_element_type=jnp.float32)
    w_log = jnp.where(w > 0.5, jnp.log(w), NEG)

    q2 = q_ref[...]
    kr = k_ref[...].reshape(T, H, D)
    vr = v_ref[...].reshape(T, H, D)

    s_list = []
    for h in range(H):
        s_h = lax.dot_general(
            q2[h * B:(h + 1) * B, :], kr[:, h, :],
            (((1,), (1,)), ((), ())),
            preferred_element_type=jnp.float32)
        s_list.append(s_h * SCALE + w_log)
    s2 = jnp.concatenate(s_list, axis=0)

    m_prev = m_ref[...]
    m_new = jnp.maximum(m_prev, jnp.max(s2, axis=1, keepdims=True))
    alpha = jnp.exp(m_prev - m_new)
    p2 = jnp.exp(s2 - m_new)
    m_ref[...] = m_new
    l_ref[...] = alpha * l_ref[...] + jnp.sum(p2, axis=1, keepdims=True)

    o_list = []
    for h in range(H):
        o_list.append(lax.dot_general(
            p2[h * B:(h + 1) * B, :], vr[:, h, :],
            (((1,), (0,)), ((), ())),
            preferred_element_type=jnp.float32))
    o2 = jnp.concatenate(o_list, axis=0)
    acc_ref[...] = alpha * acc_ref[...] + o2

    @pl.when(i == NBLK - 1)
    def _finish():
        ml_send[0, :, :] = m_ref[...]
        ml_send[1, :, :] = l_ref[...]
        rdmas = []
        for j, nbr in enumerate(peers):
            r_a = pltpu.make_async_remote_copy(
                src_ref=acc_ref, dst_ref=acc_comm.at[j],
                send_sem=send_sems.at[2 * j], recv_sem=recv_sems.at[2 * j],
                device_id=nbr, device_id_type=pl.DeviceIdType.MESH)
            r_m = pltpu.make_async_remote_copy(
                src_ref=ml_send, dst_ref=ml_comm.at[j],
                send_sem=send_sems.at[2 * j + 1],
                recv_sem=recv_sems.at[2 * j + 1],
                device_id=nbr, device_id_type=pl.DeviceIdType.MESH)
            r_a.start()
            r_m.start()
            rdmas.append((r_a, r_m))
        for r_a, r_m in rdmas:
            r_a.wait()
            r_m.wait()

        m_f = m_ref[...]
        for j in range(3):
            m_f = jnp.maximum(m_f, ml_comm[j, 0, :, :])
        a_loc = jnp.exp(m_ref[...] - m_f)
        l_f = a_loc * l_ref[...]
        acc_f = a_loc * acc_ref[...]
        for j in range(3):
            a_j = jnp.exp(ml_comm[j, 0, :, :] - m_f)
            l_f = l_f + a_j * ml_comm[j, 1, :, :]
            acc_f = acc_f + a_j * acc_comm[j]
        out = (acc_f / l_f).reshape(H, B, D)
        out_ref[...] = jnp.transpose(out, (1, 0, 2)).reshape(B, 1, H, D)


def kernel(Q, K, V, bt, lens):
    jidx = lax.broadcasted_iota(jnp.int32, (B, NPAGES_LOCAL), 1)
    btv = jnp.where(jidx < lens[:, None], bt, -1)

    q2 = jnp.transpose(Q.reshape(B, H, D), (1, 0, 2)).reshape(HB, D)

    x_off = (lax.axis_index("x") * NBLK).astype(jnp.int32).reshape(1)

    grid_spec = pltpu.PrefetchScalarGridSpec(
        num_scalar_prefetch=1,
        grid=(NBLK,),
        in_specs=[
            pl.BlockSpec((HB, D), lambda i, xo: (0, 0)),
            pl.BlockSpec((NPB, BS, H, D), lambda i, xo: (xo[0] + i, 0, 0, 0)),
            pl.BlockSpec((NPB, BS, H, D), lambda i, xo: (xo[0] + i, 0, 0, 0)),
            pl.BlockSpec((B, NPAGES_LOCAL), lambda i, xo: (0, 0)),
        ],
        out_specs=pl.BlockSpec((B, 1, H, D), lambda i, xo: (0, 0, 0, 0)),
        scratch_shapes=[
            pltpu.VMEM((HB, D), jnp.float32),
            pltpu.VMEM((HB, 1), jnp.float32),
            pltpu.VMEM((HB, 1), jnp.float32),
            pltpu.VMEM((3, HB, D), jnp.float32),
            pltpu.VMEM((2, HB, 1), jnp.float32),
            pltpu.VMEM((3, 2, HB, 1), jnp.float32),
            pltpu.SemaphoreType.DMA((6,)),
            pltpu.SemaphoreType.DMA((6,)),
        ],
    )

    return pl.pallas_call(
        _body,
        grid_spec=grid_spec,
        out_shape=jax.ShapeDtypeStruct((B, 1, H, D), jnp.float32),
        compiler_params=pltpu.CompilerParams(
            collective_id=0,
            dimension_semantics=("arbitrary",),
        ),
    )(x_off, q2, K, V, btv)
